# Initial kernel scaffold; baseline (speedup 1.0000x reference)
import jax
import jax.numpy as jnp
from jax import lax
from jax.experimental import pallas as pl
from jax.experimental.pallas import tpu as pltpu
from jax.experimental.pallas import tpu_sc as plsc

N = 10000
E = 320000
F1 = 128
H1, D1 = 4, 32
D2 = 16
NW = 32
EPW = E // NW
CH = 128
NCH = -(-EPW // CH)
EPAD = NCH * CH
NROWS = 10016
RPS = NROWS // 16

_mesh = plsc.VectorSubcoreMesh(core_axis_name="core", subcore_axis_name="subcore")


def _proj1_body(x_ref, ws_ref, bs_ref, wd_ref, bd_ref, os_ref, od_ref):
    x = x_ref[...]
    os_ref[...] = jnp.dot(x, ws_ref[...], preferred_element_type=jnp.float32) + bs_ref[...]
    od_ref[...] = jnp.dot(x, wd_ref[...], preferred_element_type=jnp.float32) + bd_ref[...]


def _proj1(x, ws, bs, wd, bd):
    bm = 2000
    return pl.pallas_call(
        _proj1_body,
        grid=(N // bm,),
        in_specs=[
            pl.BlockSpec((bm, F1), lambda i: (i, 0)),
            pl.BlockSpec((F1, F1), lambda i: (0, 0)),
            pl.BlockSpec((1, F1), lambda i: (0, 0)),
            pl.BlockSpec((F1, F1), lambda i: (0, 0)),
            pl.BlockSpec((1, F1), lambda i: (0, 0)),
        ],
        out_specs=[
            pl.BlockSpec((bm, F1), lambda i: (i, 0)),
            pl.BlockSpec((bm, F1), lambda i: (i, 0)),
        ],
        out_shape=[
            jax.ShapeDtypeStruct((N, F1), jnp.float32),
            jax.ShapeDtypeStruct((N, F1), jnp.float32),
        ],
    )(x, ws, bs, wd, bd)


def _mid_body(m0_ref, m1_ref, d0_ref, d1_ref, ws_ref, bs_ref, wd_ref, bd_ref,
              os_ref, od_ref):
    m = m0_ref[0] + m1_ref[0]
    d = d0_ref[0] + d1_ref[0]
    parts = []
    for h in range(H1):
        den = d[:, h:h + 1] + 1e-9
        parts.append(jnp.maximum(m[:, D1 * h:D1 * (h + 1)] / den, 0.0))
    hfeat = jnp.concatenate(parts, axis=1)
    os_ref[...] = jnp.dot(hfeat, ws_ref[...], preferred_element_type=jnp.float32) + bs_ref[...]
    od_ref[...] = jnp.dot(hfeat, wd_ref[...], preferred_element_type=jnp.float32) + bd_ref[...]


def _mid(msg, den, ws, bs, wd, bd):
    bm = 1024
    g = -(-NROWS // bm)
    return pl.pallas_call(
        _mid_body,
        grid=(g,),
        in_specs=[
            pl.BlockSpec((1, bm, F1), lambda i: (0, i, 0)),
            pl.BlockSpec((1, bm, F1), lambda i: (1, i, 0)),
            pl.BlockSpec((1, bm, 16), lambda i: (0, i, 0)),
            pl.BlockSpec((1, bm, 16), lambda i: (1, i, 0)),
            pl.BlockSpec((F1, D2), lambda i: (0, 0)),
            pl.BlockSpec((1, D2), lambda i: (0, 0)),
            pl.BlockSpec((F1, D2), lambda i: (0, 0)),
            pl.BlockSpec((1, D2), lambda i: (0, 0)),
        ],
        out_specs=[
            pl.BlockSpec((bm, D2), lambda i: (i, 0)),
            pl.BlockSpec((bm, D2), lambda i: (i, 0)),
        ],
        out_shape=[
            jax.ShapeDtypeStruct((NROWS, D2), jnp.float32),
            jax.ShapeDtypeStruct((NROWS, D2), jnp.float32),
        ],
    )(msg, den, ws, bs, wd, bd)


def _final_body(m0_ref, m1_ref, d0_ref, d1_ref, o_ref):
    m = m0_ref[0] + m1_ref[0]
    d = d0_ref[0][:, 0:1] + d1_ref[0][:, 0:1] + 1e-9
    o_ref[...] = m / d


def _final(msg, den):
    bm = 1024
    g = -(-NROWS // bm)
    return pl.pallas_call(
        _final_body,
        grid=(g,),
        in_specs=[
            pl.BlockSpec((1, bm, D2), lambda i: (0, i, 0)),
            pl.BlockSpec((1, bm, D2), lambda i: (1, i, 0)),
            pl.BlockSpec((1, bm, 16), lambda i: (0, i, 0)),
            pl.BlockSpec((1, bm, 16), lambda i: (1, i, 0)),
        ],
        out_specs=pl.BlockSpec((bm, D2), lambda i: (i, 0)),
        out_shape=jax.ShapeDtypeStruct((NROWS, D2), jnp.float32),
    )(msg, den)


def _zero_rows(zrow_v, nrow, ncol):
    zeros16 = jnp.zeros((16,), jnp.float32)

    @pl.loop(0, nrow)
    def _(r):
        for k in range(ncol // 16):
            zrow_v[r, pl.ds(16 * k, 16)] = zeros16


def _zero_shared(zrow_v, acc, base):
    nfull = RPS // CH
    for j in range(nfull):
        pltpu.sync_copy(zrow_v, acc.at[pl.ds(base + j * CH, CH)])
    rem = RPS - nfull * CH
    if rem:
        pltpu.sync_copy(zrow_v.at[pl.ds(0, rem)], acc.at[pl.ds(base + nfull * CH, rem)])


def _sc_pass1_body(fs_hbm, fd_hbm, src_hbm, dst_hbm, attn_hbm,
                   msg_out, den_out,
                   src_v, dst_v, fs_v, fd_v, bufm_v, buft_v, attn_v,
                   zrow_v, zt_v, msgacc, denacc, sem_s, sem_d):
    c = lax.axis_index("core")
    s = lax.axis_index("subcore")
    wid = c * 16 + s
    base = s * RPS

    _zero_rows(zrow_v, CH, F1)
    _zero_rows(zt_v, CH, 16)
    _zero_shared(zrow_v, msgacc, base)
    _zero_shared(zt_v, denacc, base)

    pltpu.sync_copy(attn_hbm, attn_v)
    a = [(attn_v[h, pl.ds(0, 16)], attn_v[h, pl.ds(16, 16)]) for h in range(H1)]
    lane = lax.iota(jnp.int32, 16)
    zeros16 = jnp.zeros((16,), jnp.float32)

    plsc.subcore_barrier()

    @pl.loop(0, NCH)
    def _(i):
        eb = i * CH
        pltpu.sync_copy(src_hbm.at[wid, pl.ds(eb, CH)], src_v)
        pltpu.sync_copy(dst_hbm.at[wid, pl.ds(eb, CH)], dst_v)
        cp1 = pltpu.async_copy(fs_hbm.at[src_v], fs_v, sem_s)
        cp2 = pltpu.async_copy(fd_hbm.at[dst_v], fd_v, sem_d)
        cp1.wait()
        cp2.wait()

        @pl.loop(0, CH)
        def _(e):
            wvs = []
            for h in range(H1):
                f0 = fs_v[e, pl.ds(D1 * h, 16)]
                f1 = fs_v[e, pl.ds(D1 * h + 16, 16)]
                g0 = fd_v[e, pl.ds(D1 * h, 16)]
                g1 = fd_v[e, pl.ds(D1 * h + 16, 16)]
                u0 = f0 + g0
                u1 = f1 + g1
                l0 = jnp.maximum(u0, 0.2 * u0)
                l1 = jnp.maximum(u1, 0.2 * u1)
                sh = jnp.sum(l0 * a[h][0] + l1 * a[h][1])
                wv = jnp.exp(jnp.full((16,), sh, jnp.float32))
                bufm_v[e, pl.ds(D1 * h, 16)] = f0 * wv
                bufm_v[e, pl.ds(D1 * h + 16, 16)] = f1 * wv
                wvs.append(wv)
            wt = zeros16
            for h in range(H1):
                wt = jnp.where(lane == h, wvs[h], wt)
            buft_v[e, pl.ds(0, 16)] = wt

        pltpu.sync_copy(bufm_v, msgacc.at[dst_v], add=True)
        pltpu.sync_copy(buft_v, denacc.at[dst_v], add=True)

    plsc.subcore_barrier()
    pltpu.sync_copy(msgacc.at[pl.ds(base, RPS)], msg_out.at[c, pl.ds(base, RPS)])
    pltpu.sync_copy(denacc.at[pl.ds(base, RPS)], den_out.at[c, pl.ds(base, RPS)])


def _sc_edge_pass1(fs, fd, src_p, dst_p, attn):
    f = pl.kernel(
        _sc_pass1_body,
        out_type=[
            jax.ShapeDtypeStruct((2, NROWS, F1), jnp.float32),
            jax.ShapeDtypeStruct((2, NROWS, 16), jnp.float32),
        ],
        mesh=_mesh,
        scratch_types=[
            pltpu.VMEM((CH,), jnp.int32),
            pltpu.VMEM((CH,), jnp.int32),
            pltpu.VMEM((CH, F1), jnp.float32),
            pltpu.VMEM((CH, F1), jnp.float32),
            pltpu.VMEM((CH, F1), jnp.float32),
            pltpu.VMEM((CH, 16), jnp.float32),
            pltpu.VMEM((H1, D1), jnp.float32),
            pltpu.VMEM((CH, F1), jnp.float32),
            pltpu.VMEM((CH, 16), jnp.float32),
            pltpu.VMEM_SHARED((NROWS, F1), jnp.float32),
            pltpu.VMEM_SHARED((NROWS, 16), jnp.float32),
            pltpu.SemaphoreType.DMA,
            pltpu.SemaphoreType.DMA,
        ],
    )
    return f(fs, fd, src_p, dst_p, attn)


def _sc_pass2_body(fs_hbm, fd_hbm, src_hbm, dst_hbm, attn_hbm,
                   msg_out, den_out,
                   src_v, dst_v, fs_v, fd_v, bufm_v, buft_v, attn_v,
                   zt_v, msgacc, denacc, sem_s, sem_d):
    c = lax.axis_index("core")
    s = lax.axis_index("subcore")
    wid = c * 16 + s
    base = s * RPS

    _zero_rows(zt_v, CH, 16)
    _zero_shared(zt_v, msgacc, base)
    _zero_shared(zt_v, denacc, base)

    pltpu.sync_copy(attn_hbm, attn_v)
    a0 = attn_v[0, pl.ds(0, 16)]
    lane = lax.iota(jnp.int32, 16)
    zeros16 = jnp.zeros((16,), jnp.float32)

    plsc.subcore_barrier()

    @pl.loop(0, NCH)
    def _(i):
        eb = i * CH
        pltpu.sync_copy(src_hbm.at[wid, pl.ds(eb, CH)], src_v)
        pltpu.sync_copy(dst_hbm.at[wid, pl.ds(eb, CH)], dst_v)
        cp1 = pltpu.async_copy(fs_hbm.at[src_v], fs_v, sem_s)
        cp2 = pltpu.async_copy(fd_hbm.at[dst_v], fd_v, sem_d)
        cp1.wait()
        cp2.wait()

        @pl.loop(0, CH)
        def _(e):
            f0 = fs_v[e, pl.ds(0, 16)]
            g0 = fd_v[e, pl.ds(0, 16)]
            u0 = f0 + g0
            l0 = jnp.maximum(u0, 0.2 * u0)
            sh = jnp.sum(l0 * a0)
            wv = jnp.exp(jnp.full((16,), sh, jnp.float32))
            bufm_v[e, pl.ds(0, 16)] = f0 * wv
            buft_v[e, pl.ds(0, 16)] = jnp.where(lane == 0, wv, zeros16)

        pltpu.sync_copy(bufm_v, msgacc.at[dst_v], add=True)
        pltpu.sync_copy(buft_v, denacc.at[dst_v], add=True)

    plsc.subcore_barrier()
    pltpu.sync_copy(msgacc.at[pl.ds(base, RPS)], msg_out.at[c, pl.ds(base, RPS)])
    pltpu.sync_copy(denacc.at[pl.ds(base, RPS)], den_out.at[c, pl.ds(base, RPS)])


def _sc_edge_pass2(fs, fd, src_p, dst_p, attn):
    f = pl.kernel(
        _sc_pass2_body,
        out_type=[
            jax.ShapeDtypeStruct((2, NROWS, D2), jnp.float32),
            jax.ShapeDtypeStruct((2, NROWS, 16), jnp.float32),
        ],
        mesh=_mesh,
        scratch_types=[
            pltpu.VMEM((CH,), jnp.int32),
            pltpu.VMEM((CH,), jnp.int32),
            pltpu.VMEM((CH, D2), jnp.float32),
            pltpu.VMEM((CH, D2), jnp.float32),
            pltpu.VMEM((CH, D2), jnp.float32),
            pltpu.VMEM((CH, 16), jnp.float32),
            pltpu.VMEM((1, D2), jnp.float32),
            pltpu.VMEM((CH, 16), jnp.float32),
            pltpu.VMEM_SHARED((NROWS, D2), jnp.float32),
            pltpu.VMEM_SHARED((NROWS, 16), jnp.float32),
            pltpu.SemaphoreType.DMA,
            pltpu.SemaphoreType.DMA,
        ],
    )
    return f(fs, fd, src_p, dst_p, attn)


def kernel(in_feat, edge_index, W1_src, b1_src, W1_dst, b1_dst, attn1,
           W2_src, b2_src, W2_dst, b2_dst, attn2):
    src = edge_index[0].reshape(NW, EPW)
    dst = edge_index[1].reshape(NW, EPW)
    pad = EPAD - EPW
    src_p = jnp.pad(src, ((0, 0), (0, pad)), constant_values=0)
    dst_p = jnp.pad(dst, ((0, 0), (0, pad)), constant_values=N)

    fs1, fd1 = _proj1(in_feat, W1_src, b1_src.reshape(1, F1),
                      W1_dst, b1_dst.reshape(1, F1))
    msg1, den1 = _sc_edge_pass1(fs1, fd1, src_p, dst_p, attn1)
    fs2, fd2 = _mid(msg1, den1, W2_src, b2_src.reshape(1, D2),
                    W2_dst, b2_dst.reshape(1, D2))
    msg2, den2 = _sc_edge_pass2(fs2, fd2, src_p, dst_p, attn2)
    out = _final(msg2, den2)
    return out[:N]


# speedup vs baseline: 26.1127x; 26.1127x over previous
import dataclasses

import jax
import jax.numpy as jnp
from jax import lax
from jax.experimental import pallas as pl
from jax.experimental.pallas import tpu as pltpu
from jax.experimental.pallas import tpu_sc as plsc

N = 10000
E = 320000
F1 = 128
H1, D1 = 4, 32
D2 = 16
NW = 32
EPW = E // NW
CH = 64
NCH = -(-EPW // CH)
EPAD = NCH * CH
NROWS = 10112
RPS = NROWS // 16
GD1 = 1280
RPD1 = GD1 // 16
GD2 = 2560
RPD2 = GD2 // 16

_mesh = plsc.VectorSubcoreMesh(core_axis_name="core", subcore_axis_name="subcore")

_sc_params = pltpu.CompilerParams()
if "needs_layout_passes" in pltpu.CompilerParams.__dataclass_fields__:
    _sc_params = dataclasses.replace(_sc_params, needs_layout_passes=False)


def _proj1_body(x_ref, ws_ref, bs_ref, wd_ref, bd_ref, os_ref, od_ref):
    x = x_ref[...]
    os_ref[...] = jnp.dot(x, ws_ref[...], preferred_element_type=jnp.float32) + bs_ref[...]
    od_ref[...] = jnp.dot(x, wd_ref[...], preferred_element_type=jnp.float32) + bd_ref[...]


def _proj1(x, ws, bs, wd, bd):
    bm = 1264
    return pl.pallas_call(
        _proj1_body,
        grid=(NROWS // bm,),
        in_specs=[
            pl.BlockSpec((bm, F1), lambda i: (i, 0)),
            pl.BlockSpec((F1, F1), lambda i: (0, 0)),
            pl.BlockSpec((1, F1), lambda i: (0, 0)),
            pl.BlockSpec((F1, F1), lambda i: (0, 0)),
            pl.BlockSpec((1, F1), lambda i: (0, 0)),
        ],
        out_specs=[
            pl.BlockSpec((bm, F1), lambda i: (i, 0)),
            pl.BlockSpec((bm, F1), lambda i: (i, 0)),
        ],
        out_shape=[
            jax.ShapeDtypeStruct((NROWS, F1), jnp.float32),
            jax.ShapeDtypeStruct((NROWS, F1), jnp.float32),
        ],
    )(x, ws, bs, wd, bd)


def _mid_body(m0_ref, m1_ref, d0_ref, d1_ref, ws_ref, bs_ref, wd_ref, bd_ref,
              os_ref, od_ref):
    m = m0_ref[0] + m1_ref[0]
    d = d0_ref[0] + d1_ref[0]
    parts = []
    for h in range(H1):
        den = d[:, h:h + 1] + 1e-9
        parts.append(jnp.maximum(m[:, D1 * h:D1 * (h + 1)] / den, 0.0))
    hfeat = jnp.concatenate(parts, axis=1)
    rs = jnp.dot(hfeat, ws_ref[...], preferred_element_type=jnp.float32) + bs_ref[...]
    rd = jnp.dot(hfeat, wd_ref[...], preferred_element_type=jnp.float32) + bd_ref[...]
    z = jnp.zeros_like(rs)
    os_ref[...] = jnp.concatenate([rs, z, z, z, z, z, z, z], axis=1)
    od_ref[...] = jnp.concatenate([rd, z, z, z, z, z, z, z], axis=1)


def _mid(msg, den, ws, bs, wd, bd):
    bm = 1024
    g = -(-NROWS // bm)
    return pl.pallas_call(
        _mid_body,
        grid=(g,),
        in_specs=[
            pl.BlockSpec((1, bm, F1), lambda i: (0, i, 0)),
            pl.BlockSpec((1, bm, F1), lambda i: (1, i, 0)),
            pl.BlockSpec((1, bm, 16), lambda i: (0, i, 0)),
            pl.BlockSpec((1, bm, 16), lambda i: (1, i, 0)),
            pl.BlockSpec((F1, D2), lambda i: (0, 0)),
            pl.BlockSpec((1, D2), lambda i: (0, 0)),
            pl.BlockSpec((F1, D2), lambda i: (0, 0)),
            pl.BlockSpec((1, D2), lambda i: (0, 0)),
        ],
        out_specs=[
            pl.BlockSpec((bm, F1), lambda i: (i, 0)),
            pl.BlockSpec((bm, F1), lambda i: (i, 0)),
        ],
        out_shape=[
            jax.ShapeDtypeStruct((NROWS, F1), jnp.float32),
            jax.ShapeDtypeStruct((NROWS, F1), jnp.float32),
        ],
    )(msg, msg, den, den, ws, bs, wd, bd)


def _final_body(md0_ref, md1_ref, o_ref):
    md = md0_ref[0] + md1_ref[0]
    d = md[:, D2:D2 + 1] + 1e-9
    o_ref[...] = md[:, :D2] / d


def _final(msgden):
    bm = 1024
    g = -(-NROWS // bm)
    return pl.pallas_call(
        _final_body,
        grid=(g,),
        in_specs=[
            pl.BlockSpec((1, bm, 32), lambda i: (0, i, 0)),
            pl.BlockSpec((1, bm, 32), lambda i: (1, i, 0)),
        ],
        out_specs=pl.BlockSpec((bm, D2), lambda i: (i, 0)),
        out_shape=jax.ShapeDtypeStruct((NROWS, D2), jnp.float32),
    )(msgden, msgden)


def _zero_rows(buf_v, nrow):
    zeros16 = jnp.zeros((16,), jnp.float32)

    @pl.loop(0, nrow)
    def _(r):
        for k in range(8):
            buf_v[r, pl.ds(16 * k, 16)] = zeros16


def _zero_shared(zrow_v, acc, base, rows):
    nfull = rows // CH
    for j in range(nfull):
        pltpu.sync_copy(zrow_v, acc.at[pl.ds(base + j * CH, CH)])
    rem = rows - nfull * CH
    if rem:
        pltpu.sync_copy(zrow_v.at[pl.ds(0, rem)], acc.at[pl.ds(base + nfull * CH, rem)])


def _dump_shared(acc, out, c, base, rows, bounce_v):
    nfull = rows // CH
    for j in range(nfull + 1):
        r = base + j * CH
        w = CH if j < nfull else rows - nfull * CH
        if w == 0:
            break
        pltpu.sync_copy(acc.at[pl.ds(r, w)], bounce_v.at[pl.ds(0, w)])
        pltpu.sync_copy(bounce_v.at[pl.ds(0, w)], out.at[c, pl.ds(r, w)])


def _shift_idx(idx_v, out_v, sh):
    for k in range(CH // 16):
        out_v[pl.ds(16 * k, 16)] = lax.shift_right_logical(idx_v[pl.ds(16 * k, 16)], sh)


def _sc_pass1_body(fs_hbm, fd_hbm, src_hbm, dst_hbm, attn_hbm,
                   msg_out, den_out,
                   src_v, dst_v, dstq_v, q_v, fs_v, fd_v, bufm_v, buft_v, attn_v,
                   msgacc, denacc, sem_s, sem_d):
    c = lax.axis_index("core")
    s = lax.axis_index("subcore")
    wid = c * 16 + s

    _zero_rows(bufm_v, CH)
    _zero_shared(bufm_v, msgacc, s * RPS, RPS)
    _zero_shared(bufm_v, denacc, s * RPD1, RPD1)

    pltpu.sync_copy(attn_hbm, attn_v)
    a = [(attn_v[0, pl.ds(D1 * h, 16)], attn_v[0, pl.ds(D1 * h + 16, 16)])
         for h in range(H1)]
    lane = lax.iota(jnp.int32, 16)
    zeros16 = jnp.zeros((16,), jnp.float32)

    plsc.subcore_barrier()

    @pl.loop(0, NCH)
    def _(i):
        pltpu.sync_copy(src_hbm.at[wid, i, pl.ds(0, CH)], src_v)
        pltpu.sync_copy(dst_hbm.at[wid, i, pl.ds(0, CH)], dst_v)
        cp1 = pltpu.async_copy(fs_hbm.at[src_v], fs_v, sem_s)
        cp2 = pltpu.async_copy(fd_hbm.at[dst_v], fd_v, sem_d)
        cp1.wait()
        cp2.wait()
        _shift_idx(dst_v, dstq_v, 3)
        for k in range(CH // 16):
            q_v[pl.ds(16 * k, 16)] = (dst_v[pl.ds(16 * k, 16)] & 7) * 16

        @pl.loop(0, CH)
        def _(e):
            wvs = []
            for h in range(H1):
                f0 = fs_v[e, pl.ds(D1 * h, 16)]
                f1 = fs_v[e, pl.ds(D1 * h + 16, 16)]
                g0 = fd_v[e, pl.ds(D1 * h, 16)]
                g1 = fd_v[e, pl.ds(D1 * h + 16, 16)]
                u0 = f0 + g0
                u1 = f1 + g1
                l0 = jnp.maximum(u0, 0.2 * u0)
                l1 = jnp.maximum(u1, 0.2 * u1)
                sh = jnp.sum(l0 * a[h][0] + l1 * a[h][1])
                wv = jnp.exp(jnp.full((16,), sh, jnp.float32))
                bufm_v[e, pl.ds(D1 * h, 16)] = f0 * wv
                bufm_v[e, pl.ds(D1 * h + 16, 16)] = f1 * wv
                wvs.append(wv)
            wt = jnp.where(lane == 0, wvs[0], zeros16)
            for h in range(1, H1):
                wt = jnp.where(lane == h, wvs[h], wt)
            q16 = q_v[pl.ds(e, 16)][0]
            for k in range(8):
                buft_v[e, pl.ds(16 * k, 16)] = zeros16
            buft_v[e, pl.ds(q16, 16)] = wt

        pltpu.sync_copy(bufm_v, msgacc.at[dst_v], add=True)
        pltpu.sync_copy(buft_v, denacc.at[dstq_v], add=True)

    plsc.subcore_barrier()
    _dump_shared(msgacc, msg_out, c, s * RPS, RPS, bufm_v)
    _dump_shared(denacc, den_out, c, s * RPD1, RPD1, buft_v)


def _sc_edge_pass1(fs, fd, src_p, dst_p, attn):
    f = pl.kernel(
        _sc_pass1_body,
        out_type=[
            jax.ShapeDtypeStruct((2, NROWS, F1), jnp.float32),
            jax.ShapeDtypeStruct((2, GD1, F1), jnp.float32),
        ],
        mesh=_mesh,
        scratch_types=[
            pltpu.VMEM((CH,), jnp.int32),
            pltpu.VMEM((CH,), jnp.int32),
            pltpu.VMEM((CH,), jnp.int32),
            pltpu.VMEM((CH + 16,), jnp.int32),
            pltpu.VMEM((CH, F1), jnp.float32),
            pltpu.VMEM((CH, F1), jnp.float32),
            pltpu.VMEM((CH, F1), jnp.float32),
            pltpu.VMEM((CH, F1), jnp.float32),
            pltpu.VMEM((1, F1), jnp.float32),
            pltpu.VMEM_SHARED((NROWS, F1), jnp.float32),
            pltpu.VMEM_SHARED((GD1, F1), jnp.float32),
            pltpu.SemaphoreType.DMA,
            pltpu.SemaphoreType.DMA,
        ],
        compiler_params=_sc_params,
    )
    return f(fs, fd, src_p, dst_p, attn)


def _sc_pass2_body(fs_hbm, fd_hbm, src_hbm, dst_hbm, attn_hbm,
                   md_out,
                   src_v, dst_v, dstq_v, q_v, fs_v, fd_v, bufm_v, attn_v,
                   mdacc, sem_s, sem_d):
    c = lax.axis_index("core")
    s = lax.axis_index("subcore")
    wid = c * 16 + s

    _zero_rows(bufm_v, CH)
    _zero_shared(bufm_v, mdacc, s * RPD2, RPD2)

    pltpu.sync_copy(attn_hbm, attn_v)
    a0 = attn_v[0, pl.ds(0, 16)]
    lane = lax.iota(jnp.int32, 16)
    zeros16 = jnp.zeros((16,), jnp.float32)

    plsc.subcore_barrier()

    @pl.loop(0, NCH)
    def _(i):
        pltpu.sync_copy(src_hbm.at[wid, i, pl.ds(0, CH)], src_v)
        pltpu.sync_copy(dst_hbm.at[wid, i, pl.ds(0, CH)], dst_v)
        cp1 = pltpu.async_copy(fs_hbm.at[src_v], fs_v, sem_s)
        cp2 = pltpu.async_copy(fd_hbm.at[dst_v], fd_v, sem_d)
        cp1.wait()
        cp2.wait()
        _shift_idx(dst_v, dstq_v, 2)
        for k in range(CH // 16):
            q_v[pl.ds(16 * k, 16)] = (dst_v[pl.ds(16 * k, 16)] & 3) * 32

        @pl.loop(0, CH)
        def _(e):
            f0 = fs_v[e, pl.ds(0, 16)]
            g0 = fd_v[e, pl.ds(0, 16)]
            u0 = f0 + g0
            l0 = jnp.maximum(u0, 0.2 * u0)
            sh = jnp.sum(l0 * a0)
            wv = jnp.exp(jnp.full((16,), sh, jnp.float32))
            q32 = q_v[pl.ds(e, 16)][0]
            for k in range(8):
                bufm_v[e, pl.ds(16 * k, 16)] = zeros16
            bufm_v[e, pl.ds(q32, 16)] = f0 * wv
            bufm_v[e, pl.ds(q32 + 16, 16)] = jnp.where(lane == 0, wv, zeros16)

        pltpu.sync_copy(bufm_v, mdacc.at[dstq_v], add=True)

    plsc.subcore_barrier()
    _dump_shared(mdacc, md_out, c, s * RPD2, RPD2, bufm_v)


def _sc_edge_pass2(fs, fd, src_p, dst_p, attn):
    f = pl.kernel(
        _sc_pass2_body,
        out_type=jax.ShapeDtypeStruct((2, GD2, F1), jnp.float32),
        mesh=_mesh,
        scratch_types=[
            pltpu.VMEM((CH,), jnp.int32),
            pltpu.VMEM((CH,), jnp.int32),
            pltpu.VMEM((CH,), jnp.int32),
            pltpu.VMEM((CH + 16,), jnp.int32),
            pltpu.VMEM((CH, F1), jnp.float32),
            pltpu.VMEM((CH, F1), jnp.float32),
            pltpu.VMEM((CH, F1), jnp.float32),
            pltpu.VMEM((1, F1), jnp.float32),
            pltpu.VMEM_SHARED((GD2, F1), jnp.float32),
            pltpu.SemaphoreType.DMA,
            pltpu.SemaphoreType.DMA,
        ],
        compiler_params=_sc_params,
    )
    return f(fs, fd, src_p, dst_p, attn)


def kernel(in_feat, edge_index, W1_src, b1_src, W1_dst, b1_dst, attn1,
           W2_src, b2_src, W2_dst, b2_dst, attn2):
    src = edge_index[0].reshape(NW, EPW)
    dst = edge_index[1].reshape(NW, EPW)
    pad = EPAD - EPW
    src_p = jnp.pad(jnp.pad(src, ((0, 0), (0, pad)), constant_values=0)
                    .reshape(NW, NCH, CH), ((0, 0), (0, 0), (0, 128 - CH)),
                    constant_values=0)
    dst_p = jnp.pad(jnp.pad(dst, ((0, 0), (0, pad)), constant_values=N)
                    .reshape(NW, NCH, CH), ((0, 0), (0, 0), (0, 128 - CH)),
                    constant_values=N)
    attn1_p = attn1.reshape(1, H1 * D1)
    attn2_p = jnp.pad(attn2.reshape(1, D2), ((0, 0), (0, F1 - D2)))

    x_pad = jnp.pad(in_feat, ((0, NROWS - N), (0, 0)))
    fs1, fd1 = _proj1(x_pad, W1_src, b1_src.reshape(1, F1),
                      W1_dst, b1_dst.reshape(1, F1))
    msg1, den1g = _sc_edge_pass1(fs1, fd1, src_p, dst_p, attn1_p)
    den1 = den1g[:, :NROWS // 8, :].reshape(2, NROWS, 16)
    fs2, fd2 = _mid(msg1, den1, W2_src, b2_src.reshape(1, D2),
                    W2_dst, b2_dst.reshape(1, D2))
    md2g = _sc_edge_pass2(fs2, fd2, src_p, dst_p, attn2_p)
    md2 = md2g[:, :NROWS // 4, :].reshape(2, NROWS, 32)
    out = _final(md2)
    return out[:N]


# speedup vs baseline: 29.1505x; 1.1163x over previous
import dataclasses

import jax
import jax.numpy as jnp
from jax import lax
from jax.experimental import pallas as pl
from jax.experimental.pallas import tpu as pltpu
from jax.experimental.pallas import tpu_sc as plsc

N = 10000
E = 320000
F1 = 128
H1, D1 = 4, 32
D2 = 16
NW = 32
EPW = E // NW
CH = 48
NCH = 2 * (-(-EPW // (2 * CH)))
EPAD = NCH * CH
NROWS = 10112
RPS = NROWS // 16
GD1 = 1280
RPD1 = GD1 // 16
GD2 = 2560
RPD2 = GD2 // 16

_mesh = plsc.VectorSubcoreMesh(core_axis_name="core", subcore_axis_name="subcore")

_sc_params = pltpu.CompilerParams()
if "needs_layout_passes" in pltpu.CompilerParams.__dataclass_fields__:
    _sc_params = dataclasses.replace(_sc_params, needs_layout_passes=False)


def _proj1_body(x_ref, ws_ref, bs_ref, wd_ref, bd_ref, os_ref, od_ref):
    x = x_ref[...]
    os_ref[...] = jnp.dot(x, ws_ref[...], preferred_element_type=jnp.float32) + bs_ref[...]
    od_ref[...] = jnp.dot(x, wd_ref[...], preferred_element_type=jnp.float32) + bd_ref[...]


def _proj1(x, ws, bs, wd, bd):
    bm = 1264
    return pl.pallas_call(
        _proj1_body,
        grid=(NROWS // bm,),
        in_specs=[
            pl.BlockSpec((bm, F1), lambda i: (i, 0)),
            pl.BlockSpec((F1, F1), lambda i: (0, 0)),
            pl.BlockSpec((1, F1), lambda i: (0, 0)),
            pl.BlockSpec((F1, F1), lambda i: (0, 0)),
            pl.BlockSpec((1, F1), lambda i: (0, 0)),
        ],
        out_specs=[
            pl.BlockSpec((bm, F1), lambda i: (i, 0)),
            pl.BlockSpec((bm, F1), lambda i: (i, 0)),
        ],
        out_shape=[
            jax.ShapeDtypeStruct((NROWS, F1), jnp.float32),
            jax.ShapeDtypeStruct((NROWS, F1), jnp.float32),
        ],
    )(x, ws, bs, wd, bd)


def _mid_body(m0_ref, m1_ref, d0_ref, d1_ref, ws_ref, bs_ref, wd_ref, bd_ref,
              os_ref, od_ref):
    m = m0_ref[0] + m1_ref[0]
    d = d0_ref[0] + d1_ref[0]
    parts = []
    for h in range(H1):
        den = d[:, h:h + 1] + 1e-9
        parts.append(jnp.maximum(m[:, D1 * h:D1 * (h + 1)] / den, 0.0))
    hfeat = jnp.concatenate(parts, axis=1)
    rs = jnp.dot(hfeat, ws_ref[...], preferred_element_type=jnp.float32) + bs_ref[...]
    rd = jnp.dot(hfeat, wd_ref[...], preferred_element_type=jnp.float32) + bd_ref[...]
    z = jnp.zeros_like(rs)
    os_ref[...] = jnp.concatenate([rs, z, z, z, z, z, z, z], axis=1)
    od_ref[...] = jnp.concatenate([rd, z, z, z, z, z, z, z], axis=1)


def _mid(msg, den, ws, bs, wd, bd):
    bm = 1024
    g = -(-NROWS // bm)
    return pl.pallas_call(
        _mid_body,
        grid=(g,),
        in_specs=[
            pl.BlockSpec((1, bm, F1), lambda i: (0, i, 0)),
            pl.BlockSpec((1, bm, F1), lambda i: (1, i, 0)),
            pl.BlockSpec((1, bm, 16), lambda i: (0, i, 0)),
            pl.BlockSpec((1, bm, 16), lambda i: (1, i, 0)),
            pl.BlockSpec((F1, D2), lambda i: (0, 0)),
            pl.BlockSpec((1, D2), lambda i: (0, 0)),
            pl.BlockSpec((F1, D2), lambda i: (0, 0)),
            pl.BlockSpec((1, D2), lambda i: (0, 0)),
        ],
        out_specs=[
            pl.BlockSpec((bm, F1), lambda i: (i, 0)),
            pl.BlockSpec((bm, F1), lambda i: (i, 0)),
        ],
        out_shape=[
            jax.ShapeDtypeStruct((NROWS, F1), jnp.float32),
            jax.ShapeDtypeStruct((NROWS, F1), jnp.float32),
        ],
    )(msg, msg, den, den, ws, bs, wd, bd)


def _final_body(md0_ref, md1_ref, o_ref):
    md = md0_ref[0] + md1_ref[0]
    d = md[:, D2:D2 + 1] + 1e-9
    o_ref[...] = md[:, :D2] / d


def _final(msgden):
    bm = 1024
    g = -(-NROWS // bm)
    return pl.pallas_call(
        _final_body,
        grid=(g,),
        in_specs=[
            pl.BlockSpec((1, bm, 32), lambda i: (0, i, 0)),
            pl.BlockSpec((1, bm, 32), lambda i: (1, i, 0)),
        ],
        out_specs=pl.BlockSpec((bm, D2), lambda i: (i, 0)),
        out_shape=jax.ShapeDtypeStruct((NROWS, D2), jnp.float32),
    )(msgden, msgden)


def _zero_rows(buf_v, nrow):
    zeros16 = jnp.zeros((16,), jnp.float32)

    @pl.loop(0, nrow)
    def _(r):
        for k in range(8):
            buf_v[r, pl.ds(16 * k, 16)] = zeros16


def _zero_shared(zrow_v, acc, base, rows):
    nfull = rows // CH
    for j in range(nfull):
        pltpu.sync_copy(zrow_v, acc.at[pl.ds(base + j * CH, CH)])
    rem = rows - nfull * CH
    if rem:
        pltpu.sync_copy(zrow_v.at[pl.ds(0, rem)], acc.at[pl.ds(base + nfull * CH, rem)])


def _dump_shared(acc, out, c, base, rows, bounce_v):
    nfull = rows // CH
    for j in range(nfull + 1):
        r = base + j * CH
        w = CH if j < nfull else rows - nfull * CH
        if w == 0:
            break
        pltpu.sync_copy(acc.at[pl.ds(r, w)], bounce_v.at[pl.ds(0, w)])
        pltpu.sync_copy(bounce_v.at[pl.ds(0, w)], out.at[c, pl.ds(r, w)])


def _shift_idx(idx_v, out_v, sh):
    for k in range(CH // 16):
        out_v[pl.ds(16 * k, 16)] = lax.shift_right_logical(idx_v[pl.ds(16 * k, 16)], sh)


def _sc_pass1_body(fs_hbm, fd_hbm, src_hbm, dst_hbm, attn_hbm,
                   msg_out, den_out,
                   src_a, dst_a, dstq_a, q_a, fs_a, fd_a,
                   src_b, dst_b, dstq_b, q_b, fs_b, fd_b,
                   bufm_v, buft_v, attn_v,
                   msgacc, denacc, sem_sa, sem_da, sem_sb, sem_db):
    c = lax.axis_index("core")
    s = lax.axis_index("subcore")
    wid = c * 16 + s

    _zero_rows(bufm_v, CH)
    _zero_shared(bufm_v, msgacc, s * RPS, RPS)
    _zero_shared(bufm_v, denacc, s * RPD1, RPD1)

    pltpu.sync_copy(attn_hbm, attn_v)
    a = [(attn_v[0, pl.ds(D1 * h, 16)], attn_v[0, pl.ds(D1 * h + 16, 16)])
         for h in range(H1)]
    lane = lax.iota(jnp.int32, 16)
    zeros16 = jnp.zeros((16,), jnp.float32)

    plsc.subcore_barrier()

    def fire(i, sv, dv, dqv, qv, fv, gv, ss, sd):
        pltpu.sync_copy(src_hbm.at[wid, i, pl.ds(0, CH)], sv)
        pltpu.sync_copy(dst_hbm.at[wid, i, pl.ds(0, CH)], dv)
        pltpu.async_copy(fs_hbm.at[sv], fv, ss)
        pltpu.async_copy(fd_hbm.at[dv], gv, sd)
        _shift_idx(dv, dqv, 3)
        for k in range(CH // 16):
            qv[pl.ds(16 * k, 16)] = (dv[pl.ds(16 * k, 16)] & 7) * 16

    def process(sv, dv, dqv, qv, fv, gv, ss, sd):
        pltpu.make_async_copy(fs_hbm.at[sv], fv, ss).wait()
        pltpu.make_async_copy(fd_hbm.at[dv], gv, sd).wait()

        @pl.loop(0, CH)
        def _(e):
            wvs = []
            for h in range(H1):
                f0 = fv[e, pl.ds(D1 * h, 16)]
                f1 = fv[e, pl.ds(D1 * h + 16, 16)]
                g0 = gv[e, pl.ds(D1 * h, 16)]
                g1 = gv[e, pl.ds(D1 * h + 16, 16)]
                u0 = f0 + g0
                u1 = f1 + g1
                l0 = jnp.maximum(u0, 0.2 * u0)
                l1 = jnp.maximum(u1, 0.2 * u1)
                sh = jnp.sum(l0 * a[h][0] + l1 * a[h][1])
                wv = jnp.exp(jnp.full((16,), sh, jnp.float32))
                bufm_v[e, pl.ds(D1 * h, 16)] = f0 * wv
                bufm_v[e, pl.ds(D1 * h + 16, 16)] = f1 * wv
                wvs.append(wv)
            wt = jnp.where(lane == 0, wvs[0], zeros16)
            for h in range(1, H1):
                wt = jnp.where(lane == h, wvs[h], wt)
            q16 = qv[pl.ds(e, 16)][0]
            for k in range(8):
                buft_v[e, pl.ds(16 * k, 16)] = zeros16
            buft_v[e, pl.ds(q16, 16)] = wt

        pltpu.sync_copy(bufm_v, msgacc.at[dv], add=True)
        pltpu.sync_copy(buft_v, denacc.at[dqv], add=True)

    A = (src_a, dst_a, dstq_a, q_a, fs_a, fd_a, sem_sa, sem_da)
    B = (src_b, dst_b, dstq_b, q_b, fs_b, fd_b, sem_sb, sem_db)

    fire(0, *A)

    @pl.loop(0, NCH - 2, step=2)
    def _(i):
        fire(i + 1, *B)
        process(*A)
        fire(i + 2, *A)
        process(*B)

    fire(NCH - 1, *B)
    process(*A)
    process(*B)

    plsc.subcore_barrier()
    _dump_shared(msgacc, msg_out, c, s * RPS, RPS, bufm_v)
    _dump_shared(denacc, den_out, c, s * RPD1, RPD1, buft_v)


def _sc_edge_pass1(fs, fd, src_p, dst_p, attn):
    f = pl.kernel(
        _sc_pass1_body,
        out_type=[
            jax.ShapeDtypeStruct((2, NROWS, F1), jnp.float32),
            jax.ShapeDtypeStruct((2, GD1, F1), jnp.float32),
        ],
        mesh=_mesh,
        scratch_types=(
            [pltpu.VMEM((CH,), jnp.int32),
             pltpu.VMEM((CH,), jnp.int32),
             pltpu.VMEM((CH,), jnp.int32),
             pltpu.VMEM((CH + 16,), jnp.int32),
             pltpu.VMEM((CH, F1), jnp.float32),
             pltpu.VMEM((CH, F1), jnp.float32)] * 2 +
            [pltpu.VMEM((CH, F1), jnp.float32),
             pltpu.VMEM((CH, F1), jnp.float32),
             pltpu.VMEM((1, F1), jnp.float32),
             pltpu.VMEM_SHARED((NROWS, F1), jnp.float32),
             pltpu.VMEM_SHARED((GD1, F1), jnp.float32),
             pltpu.SemaphoreType.DMA,
             pltpu.SemaphoreType.DMA,
             pltpu.SemaphoreType.DMA,
             pltpu.SemaphoreType.DMA]
        ),
        compiler_params=_sc_params,
    )
    return f(fs, fd, src_p, dst_p, attn)


def _sc_pass2_body(fs_hbm, fd_hbm, src_hbm, dst_hbm, attn_hbm,
                   md_out,
                   src_a, dst_a, dstq_a, q_a, fs_a, fd_a,
                   src_b, dst_b, dstq_b, q_b, fs_b, fd_b,
                   bufm_v, attn_v,
                   mdacc, sem_sa, sem_da, sem_sb, sem_db):
    c = lax.axis_index("core")
    s = lax.axis_index("subcore")
    wid = c * 16 + s

    _zero_rows(bufm_v, CH)
    _zero_shared(bufm_v, mdacc, s * RPD2, RPD2)

    pltpu.sync_copy(attn_hbm, attn_v)
    a0 = attn_v[0, pl.ds(0, 16)]
    lane = lax.iota(jnp.int32, 16)
    zeros16 = jnp.zeros((16,), jnp.float32)

    plsc.subcore_barrier()

    def fire(i, sv, dv, dqv, qv, fv, gv, ss, sd):
        pltpu.sync_copy(src_hbm.at[wid, i, pl.ds(0, CH)], sv)
        pltpu.sync_copy(dst_hbm.at[wid, i, pl.ds(0, CH)], dv)
        pltpu.async_copy(fs_hbm.at[sv], fv, ss)
        pltpu.async_copy(fd_hbm.at[dv], gv, sd)
        _shift_idx(dv, dqv, 2)
        for k in range(CH // 16):
            qv[pl.ds(16 * k, 16)] = (dv[pl.ds(16 * k, 16)] & 3) * 32

    def process(sv, dv, dqv, qv, fv, gv, ss, sd):
        pltpu.make_async_copy(fs_hbm.at[sv], fv, ss).wait()
        pltpu.make_async_copy(fd_hbm.at[dv], gv, sd).wait()

        @pl.loop(0, CH)
        def _(e):
            f0 = fv[e, pl.ds(0, 16)]
            g0 = gv[e, pl.ds(0, 16)]
            u0 = f0 + g0
            l0 = jnp.maximum(u0, 0.2 * u0)
            sh = jnp.sum(l0 * a0)
            wv = jnp.exp(jnp.full((16,), sh, jnp.float32))
            q32 = qv[pl.ds(e, 16)][0]
            for k in range(8):
                bufm_v[e, pl.ds(16 * k, 16)] = zeros16
            bufm_v[e, pl.ds(q32, 16)] = f0 * wv
            bufm_v[e, pl.ds(q32 + 16, 16)] = jnp.where(lane == 0, wv, zeros16)

        pltpu.sync_copy(bufm_v, mdacc.at[dqv], add=True)

    A = (src_a, dst_a, dstq_a, q_a, fs_a, fd_a, sem_sa, sem_da)
    B = (src_b, dst_b, dstq_b, q_b, fs_b, fd_b, sem_sb, sem_db)

    fire(0, *A)

    @pl.loop(0, NCH - 2, step=2)
    def _(i):
        fire(i + 1, *B)
        process(*A)
        fire(i + 2, *A)
        process(*B)

    fire(NCH - 1, *B)
    process(*A)
    process(*B)

    plsc.subcore_barrier()
    _dump_shared(mdacc, md_out, c, s * RPD2, RPD2, bufm_v)


def _sc_edge_pass2(fs, fd, src_p, dst_p, attn):
    f = pl.kernel(
        _sc_pass2_body,
        out_type=jax.ShapeDtypeStruct((2, GD2, F1), jnp.float32),
        mesh=_mesh,
        scratch_types=(
            [pltpu.VMEM((CH,), jnp.int32),
             pltpu.VMEM((CH,), jnp.int32),
             pltpu.VMEM((CH,), jnp.int32),
             pltpu.VMEM((CH + 16,), jnp.int32),
             pltpu.VMEM((CH, F1), jnp.float32),
             pltpu.VMEM((CH, F1), jnp.float32)] * 2 +
            [pltpu.VMEM((CH, F1), jnp.float32),
             pltpu.VMEM((1, F1), jnp.float32),
             pltpu.VMEM_SHARED((GD2, F1), jnp.float32),
             pltpu.SemaphoreType.DMA,
             pltpu.SemaphoreType.DMA,
             pltpu.SemaphoreType.DMA,
             pltpu.SemaphoreType.DMA]
        ),
        compiler_params=_sc_params,
    )
    return f(fs, fd, src_p, dst_p, attn)


def kernel(in_feat, edge_index, W1_src, b1_src, W1_dst, b1_dst, attn1,
           W2_src, b2_src, W2_dst, b2_dst, attn2):
    src = edge_index[0].reshape(NW, EPW)
    dst = edge_index[1].reshape(NW, EPW)
    pad = EPAD - EPW
    src_p = jnp.pad(jnp.pad(src, ((0, 0), (0, pad)), constant_values=0)
                    .reshape(NW, NCH, CH), ((0, 0), (0, 0), (0, 128 - CH)),
                    constant_values=0)
    dst_p = jnp.pad(jnp.pad(dst, ((0, 0), (0, pad)), constant_values=N)
                    .reshape(NW, NCH, CH), ((0, 0), (0, 0), (0, 128 - CH)),
                    constant_values=N)
    attn1_p = attn1.reshape(1, H1 * D1)
    attn2_p = jnp.pad(attn2.reshape(1, D2), ((0, 0), (0, F1 - D2)))

    x_pad = jnp.pad(in_feat, ((0, NROWS - N), (0, 0)))
    fs1, fd1 = _proj1(x_pad, W1_src, b1_src.reshape(1, F1),
                      W1_dst, b1_dst.reshape(1, F1))
    msg1, den1g = _sc_edge_pass1(fs1, fd1, src_p, dst_p, attn1_p)
    den1 = den1g[:, :NROWS // 8, :].reshape(2, NROWS, 16)
    fs2, fd2 = _mid(msg1, den1, W2_src, b2_src.reshape(1, D2),
                    W2_dst, b2_dst.reshape(1, D2))
    md2g = _sc_edge_pass2(fs2, fd2, src_p, dst_p, attn2_p)
    md2 = md2g[:, :NROWS // 4, :].reshape(2, NROWS, 32)
    out = _final(md2)
    return out[:N]


# speedup vs baseline: 43.5499x; 1.4940x over previous
import dataclasses

import jax
import jax.numpy as jnp
from jax import lax
from jax.experimental import pallas as pl
from jax.experimental.pallas import tpu as pltpu
from jax.experimental.pallas import tpu_sc as plsc

N = 10000
E = 320000
F1 = 128
H1, D1 = 4, 32
D2 = 16
NW = 32
EPW = E // NW
CH = 48
NCH = 2 * (-(-EPW // (2 * CH)))
EPAD = NCH * CH
NROWS = 10112
RPS = NROWS // 16
GD1 = 1280
RPD1 = GD1 // 16
GD2 = NROWS
RPD2 = GD2 // 16

_mesh = plsc.VectorSubcoreMesh(core_axis_name="core", subcore_axis_name="subcore")

_sc_params = pltpu.CompilerParams()
if "needs_layout_passes" in pltpu.CompilerParams.__dataclass_fields__:
    _sc_params = dataclasses.replace(_sc_params, needs_layout_passes=False)


def _proj1_body(x_ref, ws_ref, bs_ref, wd_ref, bd_ref, os_ref, od_ref):
    x = x_ref[...]
    os_ref[...] = jnp.dot(x, ws_ref[...], preferred_element_type=jnp.float32) + bs_ref[...]
    od_ref[...] = jnp.dot(x, wd_ref[...], preferred_element_type=jnp.float32) + bd_ref[...]


def _proj1(x, ws, bs, wd, bd):
    bm = 1264
    return pl.pallas_call(
        _proj1_body,
        grid=(NROWS // bm,),
        in_specs=[
            pl.BlockSpec((bm, F1), lambda i: (i, 0)),
            pl.BlockSpec((F1, F1), lambda i: (0, 0)),
            pl.BlockSpec((1, F1), lambda i: (0, 0)),
            pl.BlockSpec((F1, F1), lambda i: (0, 0)),
            pl.BlockSpec((1, F1), lambda i: (0, 0)),
        ],
        out_specs=[
            pl.BlockSpec((bm, F1), lambda i: (i, 0)),
            pl.BlockSpec((bm, F1), lambda i: (i, 0)),
        ],
        out_shape=[
            jax.ShapeDtypeStruct((NROWS, F1), jnp.float32),
            jax.ShapeDtypeStruct((NROWS, F1), jnp.float32),
        ],
    )(x, ws, bs, wd, bd)


def _mid_body(m0_ref, m1_ref, d0_ref, d1_ref, ws_ref, bs_ref, wd_ref, bd_ref,
              os_ref, od_ref):
    m = m0_ref[0] + m1_ref[0]
    d = d0_ref[0] + d1_ref[0]
    parts = []
    for h in range(H1):
        den = d[:, h:h + 1] + 1e-9
        parts.append(jnp.maximum(m[:, D1 * h:D1 * (h + 1)] / den, 0.0))
    hfeat = jnp.concatenate(parts, axis=1)
    rs = jnp.dot(hfeat, ws_ref[...], preferred_element_type=jnp.float32) + bs_ref[...]
    rd = jnp.dot(hfeat, wd_ref[...], preferred_element_type=jnp.float32) + bd_ref[...]
    z = jnp.zeros_like(rs)
    os_ref[...] = jnp.concatenate([rs, z, z, z, z, z, z, z], axis=1)
    od_ref[...] = jnp.concatenate([rd, z, z, z, z, z, z, z], axis=1)


def _mid(msg, den, ws, bs, wd, bd):
    bm = 1024
    g = -(-NROWS // bm)
    return pl.pallas_call(
        _mid_body,
        grid=(g,),
        in_specs=[
            pl.BlockSpec((1, bm, F1), lambda i: (0, i, 0)),
            pl.BlockSpec((1, bm, F1), lambda i: (1, i, 0)),
            pl.BlockSpec((1, bm, 16), lambda i: (0, i, 0)),
            pl.BlockSpec((1, bm, 16), lambda i: (1, i, 0)),
            pl.BlockSpec((F1, D2), lambda i: (0, 0)),
            pl.BlockSpec((1, D2), lambda i: (0, 0)),
            pl.BlockSpec((F1, D2), lambda i: (0, 0)),
            pl.BlockSpec((1, D2), lambda i: (0, 0)),
        ],
        out_specs=[
            pl.BlockSpec((bm, F1), lambda i: (i, 0)),
            pl.BlockSpec((bm, F1), lambda i: (i, 0)),
        ],
        out_shape=[
            jax.ShapeDtypeStruct((NROWS, F1), jnp.float32),
            jax.ShapeDtypeStruct((NROWS, F1), jnp.float32),
        ],
    )(msg, msg, den, den, ws, bs, wd, bd)


def _final_body(md0_ref, md1_ref, o_ref):
    md = md0_ref[0] + md1_ref[0]
    d = md[:, D2:D2 + 1] + 1e-9
    o_ref[...] = md[:, :D2] / d


def _final(msgden):
    bm = 1024
    g = -(-NROWS // bm)
    return pl.pallas_call(
        _final_body,
        grid=(g,),
        in_specs=[
            pl.BlockSpec((1, bm, 32), lambda i: (0, i, 0)),
            pl.BlockSpec((1, bm, 32), lambda i: (1, i, 0)),
        ],
        out_specs=pl.BlockSpec((bm, D2), lambda i: (i, 0)),
        out_shape=jax.ShapeDtypeStruct((NROWS, D2), jnp.float32),
    )(msgden, msgden)


def _zero_rows(buf_v, nrow):
    zeros16 = jnp.zeros((16,), jnp.float32)

    @pl.loop(0, nrow)
    def _(r):
        for k in range(8):
            buf_v[r, pl.ds(16 * k, 16)] = zeros16


def _zero_shared(zrow_v, acc, base, rows):
    nfull = rows // CH
    for j in range(nfull):
        pltpu.sync_copy(zrow_v, acc.at[pl.ds(base + j * CH, CH)])
    rem = rows - nfull * CH
    if rem:
        pltpu.sync_copy(zrow_v.at[pl.ds(0, rem)], acc.at[pl.ds(base + nfull * CH, rem)])


def _dump_shared(acc, out, c, base, rows, bounce_v):
    nfull = rows // CH
    for j in range(nfull + 1):
        r = base + j * CH
        w = CH if j < nfull else rows - nfull * CH
        if w == 0:
            break
        pltpu.sync_copy(acc.at[pl.ds(r, w)], bounce_v.at[pl.ds(0, w)])
        pltpu.sync_copy(bounce_v.at[pl.ds(0, w)], out.at[c, pl.ds(r, w)])


def _shift_idx(idx_v, out_v, sh):
    for k in range(CH // 16):
        out_v[pl.ds(16 * k, 16)] = lax.shift_right_logical(idx_v[pl.ds(16 * k, 16)], sh)


def _sc_pass1_body(fs_hbm, fd_hbm, src_hbm, dst_hbm, attn_hbm,
                   msg_out, den_out,
                   src_a, dst_a, dstq_a, q_a, fs_a, fd_a,
                   src_b, dst_b, dstq_b, q_b, fs_b, fd_b,
                   bufm_v, buft_v, attn_v,
                   msgacc, denacc, sem_sa, sem_da, sem_sb, sem_db):
    c = lax.axis_index("core")
    s = lax.axis_index("subcore")
    wid = c * 16 + s

    _zero_rows(bufm_v, CH)
    _zero_shared(bufm_v, msgacc, s * RPS, RPS)
    _zero_shared(bufm_v, denacc, s * RPD1, RPD1)

    pltpu.sync_copy(attn_hbm, attn_v)
    a = [(attn_v[0, pl.ds(D1 * h, 16)], attn_v[0, pl.ds(D1 * h + 16, 16)])
         for h in range(H1)]
    lane = lax.iota(jnp.int32, 16)
    zeros16 = jnp.zeros((16,), jnp.float32)

    plsc.subcore_barrier()

    def fire(i, sv, dv, dqv, qv, fv, gv, ss, sd):
        pltpu.sync_copy(src_hbm.at[wid, i, pl.ds(0, CH)], sv)
        pltpu.sync_copy(dst_hbm.at[wid, i, pl.ds(0, CH)], dv)
        pltpu.async_copy(fs_hbm.at[sv], fv, ss)
        pltpu.async_copy(fd_hbm.at[dv], gv, sd)
        _shift_idx(dv, dqv, 3)
        for k in range(CH // 16):
            qv[pl.ds(16 * k, 16)] = (dv[pl.ds(16 * k, 16)] & 7) * 16

    def process(sv, dv, dqv, qv, fv, gv, ss, sd):
        pltpu.make_async_copy(fs_hbm.at[sv], fv, ss).wait()
        pltpu.make_async_copy(fd_hbm.at[dv], gv, sd).wait()

        @plsc.parallel_loop(0, CH, unroll=2)
        def _(e):
            wvs = []
            for h in range(H1):
                f0 = fv[e, pl.ds(D1 * h, 16)]
                f1 = fv[e, pl.ds(D1 * h + 16, 16)]
                g0 = gv[e, pl.ds(D1 * h, 16)]
                g1 = gv[e, pl.ds(D1 * h + 16, 16)]
                u0 = f0 + g0
                u1 = f1 + g1
                l0 = jnp.maximum(u0, 0.2 * u0)
                l1 = jnp.maximum(u1, 0.2 * u1)
                sh = jnp.sum(l0 * a[h][0] + l1 * a[h][1])
                wv = jnp.exp(jnp.full((16,), sh, jnp.float32))
                bufm_v[e, pl.ds(D1 * h, 16)] = f0 * wv
                bufm_v[e, pl.ds(D1 * h + 16, 16)] = f1 * wv
                wvs.append(wv)
            wt = jnp.where(lane == 0, wvs[0], zeros16)
            for h in range(1, H1):
                wt = jnp.where(lane == h, wvs[h], wt)
            q16 = qv[pl.ds(e, 16)][0]
            for k in range(8):
                buft_v[e, pl.ds(16 * k, 16)] = zeros16
            buft_v[e, pl.ds(q16, 16)] = wt

        pltpu.sync_copy(bufm_v, msgacc.at[dv], add=True)
        pltpu.sync_copy(buft_v, denacc.at[dqv], add=True)

    A = (src_a, dst_a, dstq_a, q_a, fs_a, fd_a, sem_sa, sem_da)
    B = (src_b, dst_b, dstq_b, q_b, fs_b, fd_b, sem_sb, sem_db)

    fire(0, *A)

    @pl.loop(0, NCH - 2, step=2)
    def _(i):
        fire(i + 1, *B)
        process(*A)
        fire(i + 2, *A)
        process(*B)

    fire(NCH - 1, *B)
    process(*A)
    process(*B)

    plsc.subcore_barrier()
    _dump_shared(msgacc, msg_out, c, s * RPS, RPS, bufm_v)
    _dump_shared(denacc, den_out, c, s * RPD1, RPD1, buft_v)


def _sc_edge_pass1(fs, fd, src_p, dst_p, attn):
    f = pl.kernel(
        _sc_pass1_body,
        out_type=[
            jax.ShapeDtypeStruct((2, NROWS, F1), jnp.float32),
            jax.ShapeDtypeStruct((2, GD1, F1), jnp.float32),
        ],
        mesh=_mesh,
        scratch_types=(
            [pltpu.VMEM((CH,), jnp.int32),
             pltpu.VMEM((CH,), jnp.int32),
             pltpu.VMEM((CH,), jnp.int32),
             pltpu.VMEM((CH + 16,), jnp.int32),
             pltpu.VMEM((CH, F1), jnp.float32),
             pltpu.VMEM((CH, F1), jnp.float32)] * 2 +
            [pltpu.VMEM((CH, F1), jnp.float32),
             pltpu.VMEM((CH, F1), jnp.float32),
             pltpu.VMEM((1, F1), jnp.float32),
             pltpu.VMEM_SHARED((NROWS, F1), jnp.float32),
             pltpu.VMEM_SHARED((GD1, F1), jnp.float32),
             pltpu.SemaphoreType.DMA,
             pltpu.SemaphoreType.DMA,
             pltpu.SemaphoreType.DMA,
             pltpu.SemaphoreType.DMA]
        ),
        compiler_params=_sc_params,
    )
    return f(fs, fd, src_p, dst_p, attn)


def _sc_pass2_body(fs_hbm, fd_hbm, src_hbm, dst_hbm, attn_hbm,
                   md_out,
                   src_a, dst_a, dstq_a, q_a, fs_a, fd_a,
                   src_b, dst_b, dstq_b, q_b, fs_b, fd_b,
                   bufm_v, attn_v,
                   mdacc, sem_sa, sem_da, sem_sb, sem_db):
    c = lax.axis_index("core")
    s = lax.axis_index("subcore")
    wid = c * 16 + s

    _zero_rows(bufm_v, CH)
    _zero_shared(bufm_v, mdacc, s * RPD2, RPD2)

    pltpu.sync_copy(attn_hbm, attn_v)
    a0 = attn_v[0, pl.ds(0, 16)]
    lane = lax.iota(jnp.int32, 16)
    zeros16 = jnp.zeros((16,), jnp.float32)

    plsc.subcore_barrier()

    def fire(i, sv, dv, dqv, qv, fv, gv, ss, sd):
        pltpu.sync_copy(src_hbm.at[wid, i, pl.ds(0, CH)], sv)
        pltpu.sync_copy(dst_hbm.at[wid, i, pl.ds(0, CH)], dv)
        pltpu.async_copy(fs_hbm.at[sv], fv, ss)
        pltpu.async_copy(fd_hbm.at[dv], gv, sd)

    def process(sv, dv, dqv, qv, fv, gv, ss, sd):
        pltpu.make_async_copy(fs_hbm.at[sv], fv, ss).wait()
        pltpu.make_async_copy(fd_hbm.at[dv], gv, sd).wait()

        @plsc.parallel_loop(0, CH, unroll=4)
        def _(e):
            f0 = fv[e, pl.ds(0, 16)]
            g0 = gv[e, pl.ds(0, 16)]
            u0 = f0 + g0
            l0 = jnp.maximum(u0, 0.2 * u0)
            sh = jnp.sum(l0 * a0)
            wv = jnp.exp(jnp.full((16,), sh, jnp.float32))
            bufm_v[e, pl.ds(0, 16)] = f0 * wv
            bufm_v[e, pl.ds(16, 16)] = jnp.where(lane == 0, wv, zeros16)

        pltpu.sync_copy(bufm_v, mdacc.at[dv], add=True)

    A = (src_a, dst_a, dstq_a, q_a, fs_a, fd_a, sem_sa, sem_da)
    B = (src_b, dst_b, dstq_b, q_b, fs_b, fd_b, sem_sb, sem_db)

    fire(0, *A)

    @pl.loop(0, NCH - 2, step=2)
    def _(i):
        fire(i + 1, *B)
        process(*A)
        fire(i + 2, *A)
        process(*B)

    fire(NCH - 1, *B)
    process(*A)
    process(*B)

    plsc.subcore_barrier()
    _dump_shared(mdacc, md_out, c, s * RPD2, RPD2, bufm_v)


def _sc_edge_pass2(fs, fd, src_p, dst_p, attn):
    f = pl.kernel(
        _sc_pass2_body,
        out_type=jax.ShapeDtypeStruct((2, GD2, F1), jnp.float32),
        mesh=_mesh,
        scratch_types=(
            [pltpu.VMEM((CH,), jnp.int32),
             pltpu.VMEM((CH,), jnp.int32),
             pltpu.VMEM((CH,), jnp.int32),
             pltpu.VMEM((CH + 16,), jnp.int32),
             pltpu.VMEM((CH, F1), jnp.float32),
             pltpu.VMEM((CH, F1), jnp.float32)] * 2 +
            [pltpu.VMEM((CH, F1), jnp.float32),
             pltpu.VMEM((1, F1), jnp.float32),
             pltpu.VMEM_SHARED((GD2, F1), jnp.float32),
             pltpu.SemaphoreType.DMA,
             pltpu.SemaphoreType.DMA,
             pltpu.SemaphoreType.DMA,
             pltpu.SemaphoreType.DMA]
        ),
        compiler_params=_sc_params,
    )
    return f(fs, fd, src_p, dst_p, attn)


def kernel(in_feat, edge_index, W1_src, b1_src, W1_dst, b1_dst, attn1,
           W2_src, b2_src, W2_dst, b2_dst, attn2):
    src = edge_index[0].reshape(NW, EPW)
    dst = edge_index[1].reshape(NW, EPW)
    pad = EPAD - EPW
    src_p = jnp.pad(jnp.pad(src, ((0, 0), (0, pad)), constant_values=0)
                    .reshape(NW, NCH, CH), ((0, 0), (0, 0), (0, 128 - CH)),
                    constant_values=0)
    dst_p = jnp.pad(jnp.pad(dst, ((0, 0), (0, pad)), constant_values=N)
                    .reshape(NW, NCH, CH), ((0, 0), (0, 0), (0, 128 - CH)),
                    constant_values=N)
    attn1_p = attn1.reshape(1, H1 * D1)
    attn2_p = jnp.pad(attn2.reshape(1, D2), ((0, 0), (0, F1 - D2)))

    x_pad = jnp.pad(in_feat, ((0, NROWS - N), (0, 0)))
    fs1, fd1 = _proj1(x_pad, W1_src, b1_src.reshape(1, F1),
                      W1_dst, b1_dst.reshape(1, F1))
    msg1, den1g = _sc_edge_pass1(fs1, fd1, src_p, dst_p, attn1_p)
    den1 = den1g[:, :NROWS // 8, :].reshape(2, NROWS, 16)
    fs2, fd2 = _mid(msg1, den1, W2_src, b2_src.reshape(1, D2),
                    W2_dst, b2_dst.reshape(1, D2))
    md2g = _sc_edge_pass2(fs2, fd2, src_p, dst_p, attn2_p)
    md2 = md2g[:, :, :32]
    out = _final(md2)
    return out[:N]


# speedup vs baseline: 44.5873x; 1.0238x over previous
import dataclasses

import jax
import jax.numpy as jnp
from jax import lax
from jax.experimental import pallas as pl
from jax.experimental.pallas import tpu as pltpu
from jax.experimental.pallas import tpu_sc as plsc

N = 10000
E = 320000
F1 = 128
H1, D1 = 4, 32
D2 = 16
NW = 32
EPW = E // NW
CH = 48
NCH = 2 * (-(-EPW // (2 * CH)))
EPAD = NCH * CH
NROWS = 10112
RPS = NROWS // 16
GD1 = 1280
RPD1 = GD1 // 16
GD2 = NROWS
RPD2 = GD2 // 16

_mesh = plsc.VectorSubcoreMesh(core_axis_name="core", subcore_axis_name="subcore")

_sc_params = pltpu.CompilerParams()
if "needs_layout_passes" in pltpu.CompilerParams.__dataclass_fields__:
    _sc_params = dataclasses.replace(_sc_params, needs_layout_passes=False)


def _proj1_body(x_ref, ws_ref, bs_ref, wd_ref, bd_ref, os_ref, od_ref):
    x = x_ref[...]
    os_ref[...] = jnp.dot(x, ws_ref[...], preferred_element_type=jnp.float32) + bs_ref[...]
    od_ref[...] = jnp.dot(x, wd_ref[...], preferred_element_type=jnp.float32) + bd_ref[...]


def _proj1(x, ws, bs, wd, bd):
    bm = 1264
    return pl.pallas_call(
        _proj1_body,
        grid=(NROWS // bm,),
        in_specs=[
            pl.BlockSpec((bm, F1), lambda i: (i, 0)),
            pl.BlockSpec((F1, F1), lambda i: (0, 0)),
            pl.BlockSpec((1, F1), lambda i: (0, 0)),
            pl.BlockSpec((F1, F1), lambda i: (0, 0)),
            pl.BlockSpec((1, F1), lambda i: (0, 0)),
        ],
        out_specs=[
            pl.BlockSpec((bm, F1), lambda i: (i, 0)),
            pl.BlockSpec((bm, F1), lambda i: (i, 0)),
        ],
        out_shape=[
            jax.ShapeDtypeStruct((NROWS, F1), jnp.float32),
            jax.ShapeDtypeStruct((NROWS, F1), jnp.float32),
        ],
    )(x, ws, bs, wd, bd)


def _mid_body(m0_ref, m1_ref, d0_ref, d1_ref, ws_ref, bs_ref, wd_ref, bd_ref,
              os_ref, od_ref):
    m = m0_ref[0] + m1_ref[0]
    d = d0_ref[0] + d1_ref[0]
    parts = []
    for h in range(H1):
        den = d[:, h:h + 1] + 1e-9
        parts.append(jnp.maximum(m[:, D1 * h:D1 * (h + 1)] / den, 0.0))
    hfeat = jnp.concatenate(parts, axis=1)
    rs = jnp.dot(hfeat, ws_ref[...], preferred_element_type=jnp.float32) + bs_ref[...]
    rd = jnp.dot(hfeat, wd_ref[...], preferred_element_type=jnp.float32) + bd_ref[...]
    z = jnp.zeros_like(rs)
    os_ref[...] = jnp.concatenate([rs, z, z, z, z, z, z, z], axis=1)
    od_ref[...] = jnp.concatenate([rd, z, z, z, z, z, z, z], axis=1)


def _mid(msg, den, ws, bs, wd, bd):
    bm = 1024
    g = -(-NROWS // bm)
    return pl.pallas_call(
        _mid_body,
        grid=(g,),
        in_specs=[
            pl.BlockSpec((1, bm, F1), lambda i: (0, i, 0)),
            pl.BlockSpec((1, bm, F1), lambda i: (1, i, 0)),
            pl.BlockSpec((1, bm, 16), lambda i: (0, i, 0)),
            pl.BlockSpec((1, bm, 16), lambda i: (1, i, 0)),
            pl.BlockSpec((F1, D2), lambda i: (0, 0)),
            pl.BlockSpec((1, D2), lambda i: (0, 0)),
            pl.BlockSpec((F1, D2), lambda i: (0, 0)),
            pl.BlockSpec((1, D2), lambda i: (0, 0)),
        ],
        out_specs=[
            pl.BlockSpec((bm, F1), lambda i: (i, 0)),
            pl.BlockSpec((bm, F1), lambda i: (i, 0)),
        ],
        out_shape=[
            jax.ShapeDtypeStruct((NROWS, F1), jnp.float32),
            jax.ShapeDtypeStruct((NROWS, F1), jnp.float32),
        ],
    )(msg, msg, den, den, ws, bs, wd, bd)


def _final_body(md0_ref, md1_ref, o_ref):
    md = md0_ref[0] + md1_ref[0]
    d = md[:, D2:D2 + 1] + 1e-9
    o_ref[...] = md[:, :D2] / d


def _final(msgden):
    bm = 1024
    g = -(-NROWS // bm)
    return pl.pallas_call(
        _final_body,
        grid=(g,),
        in_specs=[
            pl.BlockSpec((1, bm, 32), lambda i: (0, i, 0)),
            pl.BlockSpec((1, bm, 32), lambda i: (1, i, 0)),
        ],
        out_specs=pl.BlockSpec((bm, D2), lambda i: (i, 0)),
        out_shape=jax.ShapeDtypeStruct((NROWS, D2), jnp.float32),
    )(msgden, msgden)


def _zero_rows(buf_v, nrow):
    zeros16 = jnp.zeros((16,), jnp.float32)

    @pl.loop(0, nrow)
    def _(r):
        for k in range(8):
            buf_v[r, pl.ds(16 * k, 16)] = zeros16


def _zero_shared(zrow_v, acc, base, rows):
    nfull = rows // CH
    for j in range(nfull):
        pltpu.sync_copy(zrow_v, acc.at[pl.ds(base + j * CH, CH)])
    rem = rows - nfull * CH
    if rem:
        pltpu.sync_copy(zrow_v.at[pl.ds(0, rem)], acc.at[pl.ds(base + nfull * CH, rem)])


def _dump_shared(acc, out, c, base, rows, bounce_v):
    nfull = rows // CH
    for j in range(nfull + 1):
        r = base + j * CH
        w = CH if j < nfull else rows - nfull * CH
        if w == 0:
            break
        pltpu.sync_copy(acc.at[pl.ds(r, w)], bounce_v.at[pl.ds(0, w)])
        pltpu.sync_copy(bounce_v.at[pl.ds(0, w)], out.at[c, pl.ds(r, w)])


def _shift_idx(idx_v, out_v, sh):
    for k in range(CH // 16):
        out_v[pl.ds(16 * k, 16)] = lax.shift_right_logical(idx_v[pl.ds(16 * k, 16)], sh)


def _sc_pass1_body(fs_hbm, fd_hbm, src_hbm, dst_hbm, attn_hbm,
                   msg_out, den_out,
                   src_a, dst_a, dstq_a, q_a, fs_a, fd_a,
                   src_b, dst_b, dstq_b, q_b, fs_b, fd_b,
                   bufm_v, buft_v, attn_v,
                   msgacc, denacc, sem_sa, sem_da, sem_sb, sem_db):
    c = lax.axis_index("core")
    s = lax.axis_index("subcore")
    wid = c * 16 + s

    _zero_rows(bufm_v, CH)
    _zero_shared(bufm_v, msgacc, s * RPS, RPS)
    _zero_shared(bufm_v, denacc, s * RPD1, RPD1)

    pltpu.sync_copy(attn_hbm, attn_v)
    a = [(attn_v[0, pl.ds(D1 * h, 16)], attn_v[0, pl.ds(D1 * h + 16, 16)])
         for h in range(H1)]
    lane = lax.iota(jnp.int32, 16)
    zeros16 = jnp.zeros((16,), jnp.float32)

    plsc.subcore_barrier()

    def fire(i, sv, dv, dqv, qv, fv, gv, ss, sd):
        pltpu.sync_copy(src_hbm.at[wid, i, pl.ds(0, CH)], sv)
        pltpu.sync_copy(dst_hbm.at[wid, i, pl.ds(0, CH)], dv)
        pltpu.async_copy(fs_hbm.at[sv], fv, ss)
        pltpu.async_copy(fd_hbm.at[dv], gv, sd)
        _shift_idx(dv, dqv, 3)
        for k in range(CH // 16):
            qv[pl.ds(16 * k, 16)] = (dv[pl.ds(16 * k, 16)] & 7) * 16

    def process(sv, dv, dqv, qv, fv, gv, ss, sd):
        pltpu.make_async_copy(fs_hbm.at[sv], fv, ss).wait()
        pltpu.make_async_copy(fd_hbm.at[dv], gv, sd).wait()

        @plsc.parallel_loop(0, CH, unroll=4)
        def _(e):
            wvs = []
            for h in range(H1):
                f0 = fv[e, pl.ds(D1 * h, 16)]
                f1 = fv[e, pl.ds(D1 * h + 16, 16)]
                g0 = gv[e, pl.ds(D1 * h, 16)]
                g1 = gv[e, pl.ds(D1 * h + 16, 16)]
                u0 = f0 + g0
                u1 = f1 + g1
                l0 = jnp.maximum(u0, 0.2 * u0)
                l1 = jnp.maximum(u1, 0.2 * u1)
                sh = jnp.sum(l0 * a[h][0] + l1 * a[h][1])
                wv = jnp.exp(jnp.full((16,), sh, jnp.float32))
                bufm_v[e, pl.ds(D1 * h, 16)] = f0 * wv
                bufm_v[e, pl.ds(D1 * h + 16, 16)] = f1 * wv
                wvs.append(wv)
            wt = jnp.where(lane == 0, wvs[0], zeros16)
            for h in range(1, H1):
                wt = jnp.where(lane == h, wvs[h], wt)
            q16 = qv[pl.ds(e, 16)][0]
            for k in range(8):
                buft_v[e, pl.ds(16 * k, 16)] = zeros16
            buft_v[e, pl.ds(q16, 16)] = wt

        pltpu.sync_copy(bufm_v, msgacc.at[dv], add=True)
        pltpu.sync_copy(buft_v, denacc.at[dqv], add=True)

    A = (src_a, dst_a, dstq_a, q_a, fs_a, fd_a, sem_sa, sem_da)
    B = (src_b, dst_b, dstq_b, q_b, fs_b, fd_b, sem_sb, sem_db)

    fire(0, *A)

    @pl.loop(0, NCH - 2, step=2)
    def _(i):
        fire(i + 1, *B)
        process(*A)
        fire(i + 2, *A)
        process(*B)

    fire(NCH - 1, *B)
    process(*A)
    process(*B)

    plsc.subcore_barrier()
    _dump_shared(msgacc, msg_out, c, s * RPS, RPS, bufm_v)
    _dump_shared(denacc, den_out, c, s * RPD1, RPD1, buft_v)


def _sc_edge_pass1(fs, fd, src_p, dst_p, attn):
    f = pl.kernel(
        _sc_pass1_body,
        out_type=[
            jax.ShapeDtypeStruct((2, NROWS, F1), jnp.float32),
            jax.ShapeDtypeStruct((2, GD1, F1), jnp.float32),
        ],
        mesh=_mesh,
        scratch_types=(
            [pltpu.VMEM((CH,), jnp.int32),
             pltpu.VMEM((CH,), jnp.int32),
             pltpu.VMEM((CH,), jnp.int32),
             pltpu.VMEM((CH + 16,), jnp.int32),
             pltpu.VMEM((CH, F1), jnp.float32),
             pltpu.VMEM((CH, F1), jnp.float32)] * 2 +
            [pltpu.VMEM((CH, F1), jnp.float32),
             pltpu.VMEM((CH, F1), jnp.float32),
             pltpu.VMEM((1, F1), jnp.float32),
             pltpu.VMEM_SHARED((NROWS, F1), jnp.float32),
             pltpu.VMEM_SHARED((GD1, F1), jnp.float32),
             pltpu.SemaphoreType.DMA,
             pltpu.SemaphoreType.DMA,
             pltpu.SemaphoreType.DMA,
             pltpu.SemaphoreType.DMA]
        ),
        compiler_params=_sc_params,
    )
    return f(fs, fd, src_p, dst_p, attn)


def _sc_pass2_body(fs_hbm, fd_hbm, src_hbm, dst_hbm, attn_hbm,
                   md_out,
                   src_a, dst_a, dstq_a, q_a, fs_a, fd_a,
                   src_b, dst_b, dstq_b, q_b, fs_b, fd_b,
                   bufm_v, attn_v,
                   mdacc, sem_sa, sem_da, sem_sb, sem_db):
    c = lax.axis_index("core")
    s = lax.axis_index("subcore")
    wid = c * 16 + s

    _zero_rows(bufm_v, CH)
    _zero_shared(bufm_v, mdacc, s * RPD2, RPD2)

    pltpu.sync_copy(attn_hbm, attn_v)
    a0 = attn_v[0, pl.ds(0, 16)]
    lane = lax.iota(jnp.int32, 16)
    zeros16 = jnp.zeros((16,), jnp.float32)

    plsc.subcore_barrier()

    def fire(i, sv, dv, dqv, qv, fv, gv, ss, sd):
        pltpu.sync_copy(src_hbm.at[wid, i, pl.ds(0, CH)], sv)
        pltpu.sync_copy(dst_hbm.at[wid, i, pl.ds(0, CH)], dv)
        pltpu.async_copy(fs_hbm.at[sv], fv, ss)
        pltpu.async_copy(fd_hbm.at[dv], gv, sd)

    def process(sv, dv, dqv, qv, fv, gv, ss, sd):
        pltpu.make_async_copy(fs_hbm.at[sv], fv, ss).wait()
        pltpu.make_async_copy(fd_hbm.at[dv], gv, sd).wait()

        @plsc.parallel_loop(0, CH, unroll=8)
        def _(e):
            f0 = fv[e, pl.ds(0, 16)]
            g0 = gv[e, pl.ds(0, 16)]
            u0 = f0 + g0
            l0 = jnp.maximum(u0, 0.2 * u0)
            sh = jnp.sum(l0 * a0)
            wv = jnp.exp(jnp.full((16,), sh, jnp.float32))
            bufm_v[e, pl.ds(0, 16)] = f0 * wv
            bufm_v[e, pl.ds(16, 16)] = jnp.where(lane == 0, wv, zeros16)

        pltpu.sync_copy(bufm_v, mdacc.at[dv], add=True)

    A = (src_a, dst_a, dstq_a, q_a, fs_a, fd_a, sem_sa, sem_da)
    B = (src_b, dst_b, dstq_b, q_b, fs_b, fd_b, sem_sb, sem_db)

    fire(0, *A)

    @pl.loop(0, NCH - 2, step=2)
    def _(i):
        fire(i + 1, *B)
        process(*A)
        fire(i + 2, *A)
        process(*B)

    fire(NCH - 1, *B)
    process(*A)
    process(*B)

    plsc.subcore_barrier()
    _dump_shared(mdacc, md_out, c, s * RPD2, RPD2, bufm_v)


def _sc_edge_pass2(fs, fd, src_p, dst_p, attn):
    f = pl.kernel(
        _sc_pass2_body,
        out_type=jax.ShapeDtypeStruct((2, GD2, F1), jnp.float32),
        mesh=_mesh,
        scratch_types=(
            [pltpu.VMEM((CH,), jnp.int32),
             pltpu.VMEM((CH,), jnp.int32),
             pltpu.VMEM((CH,), jnp.int32),
             pltpu.VMEM((CH + 16,), jnp.int32),
             pltpu.VMEM((CH, F1), jnp.float32),
             pltpu.VMEM((CH, F1), jnp.float32)] * 2 +
            [pltpu.VMEM((CH, F1), jnp.float32),
             pltpu.VMEM((1, F1), jnp.float32),
             pltpu.VMEM_SHARED((GD2, F1), jnp.float32),
             pltpu.SemaphoreType.DMA,
             pltpu.SemaphoreType.DMA,
             pltpu.SemaphoreType.DMA,
             pltpu.SemaphoreType.DMA]
        ),
        compiler_params=_sc_params,
    )
    return f(fs, fd, src_p, dst_p, attn)


def kernel(in_feat, edge_index, W1_src, b1_src, W1_dst, b1_dst, attn1,
           W2_src, b2_src, W2_dst, b2_dst, attn2):
    src = edge_index[0].reshape(NW, EPW)
    dst = edge_index[1].reshape(NW, EPW)
    pad = EPAD - EPW
    src_p = jnp.pad(jnp.pad(src, ((0, 0), (0, pad)), constant_values=0)
                    .reshape(NW, NCH, CH), ((0, 0), (0, 0), (0, 128 - CH)),
                    constant_values=0)
    dst_p = jnp.pad(jnp.pad(dst, ((0, 0), (0, pad)), constant_values=N)
                    .reshape(NW, NCH, CH), ((0, 0), (0, 0), (0, 128 - CH)),
                    constant_values=N)
    attn1_p = attn1.reshape(1, H1 * D1)
    attn2_p = jnp.pad(attn2.reshape(1, D2), ((0, 0), (0, F1 - D2)))

    x_pad = jnp.pad(in_feat, ((0, NROWS - N), (0, 0)))
    fs1, fd1 = _proj1(x_pad, W1_src, b1_src.reshape(1, F1),
                      W1_dst, b1_dst.reshape(1, F1))
    msg1, den1g = _sc_edge_pass1(fs1, fd1, src_p, dst_p, attn1_p)
    den1 = den1g[:, :NROWS // 8, :].reshape(2, NROWS, 16)
    fs2, fd2 = _mid(msg1, den1, W2_src, b2_src.reshape(1, D2),
                    W2_dst, b2_dst.reshape(1, D2))
    md2g = _sc_edge_pass2(fs2, fd2, src_p, dst_p, attn2_p)
    md2 = md2g[:, :, :32]
    out = _final(md2)
    return out[:N]
